# baseline (speedup 1.0000x reference)
import jax
import jax.numpy as jnp
from jax.experimental import pallas as pl
from jax.experimental.pallas import tpu as pltpu

_NC = 80
_DFL = 16
_CH = 4 * _DFL + _NC
_STRIDE = 8.0
_EPS = 1e-9
_GAIN_CLS = 0.5
_GAIN_BOX = 7.5
_GAIN_DFL = 1.5
_B, _H, _W = 16, 80, 80
_A = _H * _W
_NT = 256
_CT = 16
_NCH = _NT // _CT
_BIG = 1e4
_HIGH = jax.lax.Precision.HIGHEST


def _atan(x):
    s = jnp.sign(x)
    a = jnp.abs(x)
    big = a > 2.414213562373095
    mid = a > 0.4142135623730951
    safe = jnp.maximum(a, 1e-30)
    r = jnp.where(big, -1.0 / safe, jnp.where(mid, (a - 1.0) / (a + 1.0), a))
    y0 = jnp.where(big, jnp.float32(jnp.pi / 2),
                   jnp.where(mid, jnp.float32(jnp.pi / 4), jnp.float32(0.0)))
    z = r * r
    p = (((8.05374449538e-2 * z - 1.38776856032e-1) * z + 1.99777106478e-1)
         * z - 3.33329491539e-1) * z * r + r
    return s * (y0 + p)


def _loss_kernel(x_ref, t_ref, out_ref, gt_ref, sov_ref, al_ref):
    b = pl.program_id(0)
    f32 = jnp.float32
    tgt = t_ref[...]

    xT = jnp.transpose(x_ref[0], (1, 0))

    al_iota = jax.lax.broadcasted_iota(jnp.int32, (1, _A), 1)
    axr = (al_iota % _W).astype(f32) + 0.5
    ayr = (al_iota // _W).astype(f32) + 0.5

    proj = jax.lax.broadcasted_iota(jnp.int32, (_DFL, _A), 0).astype(f32)
    pd = []
    for c in range(4):
        ch = xT[c * _DFL:(c + 1) * _DFL, :]
        m = jnp.max(ch, axis=0, keepdims=True)
        e = jnp.exp(ch - m)
        pd.append(jnp.sum(e * proj, axis=0, keepdims=True) /
                  jnp.sum(e, axis=0, keepdims=True))
    px1 = axr - pd[0]
    py1 = ayr - pd[1]
    px2 = axr + pd[2]
    py2 = ayr + pd[3]

    psT = xT[4 * _DFL:, :]
    scoresT = jax.nn.sigmoid(psT)

    timg = tgt[:, 0:1].astype(jnp.int32)
    mask = (timg == b).astype(f32)
    ti = jax.lax.broadcasted_iota(jnp.int32, (_NT, _NT), 0)
    tj = jax.lax.broadcasted_iota(jnp.int32, (_NT, _NT), 1)
    tril = (tj <= ti).astype(f32)
    cmask = jax.lax.dot_general(tril, mask, (((1,), (0,)), ((), ())),
                                precision=_HIGH)
    occ = cmask - 1.0
    perm = mask * (tj.astype(f32) == occ)
    gt = jax.lax.dot_general(perm, tgt[:, 1:6], (((0,), (0,)), ((), ())),
                             precision=_HIGH)
    size = _W * _STRIDE
    cx = gt[:, 1:2] * size
    cy = gt[:, 2:3] * size
    bw = gt[:, 3:4] * size
    bh = gt[:, 4:5] * size
    gx1 = cx - bw * 0.5
    gy1 = cy - bh * 0.5
    gx2 = cx + bw * 0.5
    gy2 = cy + bh * 0.5
    mgt = ((gx1 + gy1 + gx2 + gy2) > 0).astype(f32)
    gt_ref[:, 0:1] = gt[:, 0:1]
    gt_ref[:, 1:2] = gx1
    gt_ref[:, 2:3] = gy1
    gt_ref[:, 3:4] = gx2
    gt_ref[:, 4:5] = gy2
    gt_ref[:, 5:6] = mgt

    bx1 = px1 * _STRIDE
    by1 = py1 * _STRIDE
    bx2 = px2 * _STRIDE
    by2 = py2 * _STRIDE
    apix = (bx2 - bx1) * (by2 - by1)
    ax8 = axr * _STRIDE
    ay8 = ayr * _STRIDE

    ciota = jax.lax.broadcasted_iota(jnp.int32, (_CT, _NC), 1).astype(f32)
    riota = jax.lax.broadcasted_iota(jnp.int32, (_CT, 1), 0).astype(f32)
    neg = jnp.float32(-jnp.inf)

    def pass1(j, carry):
        fg, best_v, best_i = carry
        row = pl.ds(pl.multiple_of(j * _CT, _CT), _CT)
        g = gt_ref[row, :]
        lblj = g[:, 0:1]
        x1j = g[:, 1:2]
        y1j = g[:, 2:3]
        x2j = g[:, 3:4]
        y2j = g[:, 4:5]
        mgtj = g[:, 5:6]
        inter = (jnp.maximum(jnp.minimum(x2j, bx2) - jnp.maximum(x1j, bx1),
                             0.0) *
                 jnp.maximum(jnp.minimum(y2j, by2) - jnp.maximum(y1j, by1),
                             0.0))
        ag = (x2j - x1j) * (y2j - y1j)
        ovj = jnp.maximum(inter / (ag + apix - inter + _EPS), 0.0)
        selj = (ciota == lblj).astype(f32)
        bscj = jax.lax.dot_general(selj, scoresT, (((1,), (0,)), ((), ())),
                                   precision=_HIGH)
        o2 = ovj * ovj
        alj = jnp.sqrt(bscj) * (o2 * o2 * o2)
        m_inj = (jnp.minimum(jnp.minimum(ax8 - x1j, ay8 - y1j),
                             jnp.minimum(x2j - ax8, y2j - ay8))
                 > _EPS).astype(f32)
        metj = alj * m_inj * mgtj

        def extract(_, work):
            mx = jnp.max(work, axis=1, keepdims=True)
            return jnp.where(work >= mx, neg, work)

        work = jax.lax.fori_loop(0, 9, extract, metj)
        kth = jnp.max(work, axis=1, keepdims=True)
        mpj = jnp.logical_and(metj >= kth, metj > _EPS)
        fg = fg + jnp.sum(mpj.astype(f32), axis=0, keepdims=True)
        ridx = riota + (j * _CT).astype(f32)
        cmx = jnp.max(ovj, axis=0, keepdims=True)
        lidx = jnp.min(jnp.where(ovj == cmx, ridx, f32(_BIG)),
                       axis=0, keepdims=True)
        better = cmx > best_v
        best_v = jnp.where(better, cmx, best_v)
        best_i = jnp.where(better, lidx, best_i)
        sov_ref[row, :] = jnp.where(mpj, ovj, -ovj)
        al_ref[row, :] = alj
        return fg, best_v, best_i

    zero_r = jnp.zeros((1, _A), f32)
    fg, best_v, best_i = jax.lax.fori_loop(
        0, _NCH, pass1, (zero_r, zero_r - 1.0, zero_r))

    def pass2(j, carry):
        fg_cnt, tgt_min, t_lbl, tx1, ty1, tx2, ty2, norm = carry
        row = pl.ds(pl.multiple_of(j * _CT, _CT), _CT)
        g = gt_ref[row, :]
        sov = sov_ref[row, :]
        ovj = jnp.abs(sov)
        mpj = (sov > 0.0).astype(f32)
        alj = al_ref[row, :]
        ridx = riota + (j * _CT).astype(f32)
        ismj = (ridx == best_i).astype(f32)
        mposj = jnp.where(fg > 1.0, ismj, mpj)
        fg_cnt = fg_cnt + jnp.sum(mposj, axis=0, keepdims=True)
        tminj = jnp.min(jnp.where(mposj > 0.5, ridx, f32(_BIG)),
                        axis=0, keepdims=True)
        qj = (ridx == tminj).astype(f32)
        upd = tminj < tgt_min
        t_lbl = jnp.where(upd, jnp.sum(qj * g[:, 0:1], axis=0,
                                       keepdims=True), t_lbl)
        tx1 = jnp.where(upd, jnp.sum(qj * g[:, 1:2], axis=0,
                                     keepdims=True), tx1)
        ty1 = jnp.where(upd, jnp.sum(qj * g[:, 2:3], axis=0,
                                     keepdims=True), ty1)
        tx2 = jnp.where(upd, jnp.sum(qj * g[:, 3:4], axis=0,
                                     keepdims=True), tx2)
        ty2 = jnp.where(upd, jnp.sum(qj * g[:, 4:5], axis=0,
                                     keepdims=True), ty2)
        tgt_min = jnp.minimum(tgt_min, tminj)
        almj = alj * mposj
        pa = jnp.max(almj, axis=1, keepdims=True)
        po = jnp.max(ovj * mposj, axis=1, keepdims=True)
        norm = jnp.maximum(norm, jnp.max(almj * po / (pa + _EPS),
                                         axis=0, keepdims=True))
        return fg_cnt, tgt_min, t_lbl, tx1, ty1, tx2, ty2, norm

    fg_cnt, tgt_min, t_lbl, tx1, ty1, tx2, ty2, norm = jax.lax.fori_loop(
        0, _NCH, pass2,
        (zero_r, zero_r + f32(_BIG), zero_r, zero_r, zero_r, zero_r, zero_r,
         zero_r))
    no_hit = tgt_min >= f32(_BIG)
    t_lbl = jnp.where(no_hit, zero_r + gt_ref[0:1, 0:1], t_lbl)
    tx1 = jnp.where(no_hit, zero_r + gt_ref[0:1, 1:2], tx1)
    ty1 = jnp.where(no_hit, zero_r + gt_ref[0:1, 2:3], ty1)
    tx2 = jnp.where(no_hit, zero_r + gt_ref[0:1, 3:4], tx2)
    ty2 = jnp.where(no_hit, zero_r + gt_ref[0:1, 4:5], ty2)
    fgm = (fg_cnt > 0).astype(f32)
    weight = norm * fgm

    base = jnp.sum(jnp.maximum(psT, 0.0) + jnp.log1p(jnp.exp(-jnp.abs(psT))),
                   keepdims=True)
    crow = jax.lax.broadcasted_iota(jnp.int32, (_NC, _A), 0).astype(f32)
    ps_at = jnp.sum(jnp.where(crow == t_lbl, psT, 0.0),
                    axis=0, keepdims=True)
    bce_sum = base - jnp.sum(weight * ps_at, keepdims=True)

    tss = jnp.sum(weight, keepdims=True)

    e7 = 1e-7
    qx1 = tx1 / _STRIDE
    qy1 = ty1 / _STRIDE
    qx2 = tx2 / _STRIDE
    qy2 = ty2 / _STRIDE
    w1 = px2 - px1
    h1 = py2 - py1
    w2 = qx2 - qx1
    h2 = qy2 - qy1
    cinter = (jnp.maximum(jnp.minimum(px2, qx2) - jnp.maximum(px1, qx1), 0.0) *
              jnp.maximum(jnp.minimum(py2, qy2) - jnp.maximum(py1, qy1), 0.0))
    cunion = w1 * h1 + w2 * h2 - cinter + e7
    ciou_i = cinter / cunion
    cw = jnp.maximum(px2, qx2) - jnp.minimum(px1, qx1)
    chh = jnp.maximum(py2, qy2) - jnp.minimum(py1, qy1)
    c2 = cw * cw + chh * chh + e7
    rho2 = ((qx1 + qx2 - px1 - px2) ** 2 + (qy1 + qy2 - py1 - py2) ** 2) / 4.0
    v = (4.0 / (jnp.pi ** 2)) * (_atan(w2 / (h2 + e7)) -
                                 _atan(w1 / (h1 + e7))) ** 2
    alpha_t = v / (v - ciou_i + 1.0 + e7)
    iou_full = ciou_i - (rho2 / c2 + v * alpha_t)
    box_sum = jnp.sum((1.0 - iou_full) * weight, keepdims=True)

    tvals = (jnp.clip(axr - qx1, 0.0, _DFL - 1.01),
             jnp.clip(ayr - qy1, 0.0, _DFL - 1.01),
             jnp.clip(qx2 - axr, 0.0, _DFL - 1.01),
             jnp.clip(qy2 - ayr, 0.0, _DFL - 1.01))
    dfl_acc = jnp.zeros((1, _A), dtype=f32)
    for c in range(4):
        ch = xT[c * _DFL:(c + 1) * _DFL, :]
        m = jnp.max(ch, axis=0, keepdims=True)
        sh = ch - m
        logsm = sh - jnp.log(jnp.sum(jnp.exp(sh), axis=0, keepdims=True))
        tv = tvals[c]
        tl = jnp.floor(tv)
        wl = (tl + 1.0) - tv
        wr = 1.0 - wl
        trc = jnp.minimum(tl + 1.0, f32(_DFL - 1))
        ce_l = -jnp.sum(jnp.where(proj == tl, logsm, 0.0),
                        axis=0, keepdims=True)
        ce_r = -jnp.sum(jnp.where(proj == trc, logsm, 0.0),
                        axis=0, keepdims=True)
        dfl_acc = dfl_acc + ce_l * wl + ce_r * wr
    dfl_sum = jnp.sum(0.25 * dfl_acc * weight, keepdims=True)

    row = jnp.concatenate([bce_sum, tss, box_sum, dfl_sum], axis=1)
    out_ref[0] = row


def kernel(outputs, targets):
    x = outputs.reshape(_B, _A, _CH)
    parts = pl.pallas_call(
        _loss_kernel,
        grid=(_B,),
        in_specs=[
            pl.BlockSpec((1, _A, _CH), lambda b: (b, 0, 0)),
            pl.BlockSpec((_NT, 6), lambda b: (0, 0)),
        ],
        out_specs=pl.BlockSpec((1, 1, 4), lambda b: (b, 0, 0)),
        out_shape=jax.ShapeDtypeStruct((_B, 1, 4), jnp.float32),
        scratch_shapes=[
            pltpu.VMEM((_NT, 8), jnp.float32),
            pltpu.VMEM((_NT, _A), jnp.float32),
            pltpu.VMEM((_NT, _A), jnp.float32),
        ],
        compiler_params=pltpu.CompilerParams(
            dimension_semantics=("arbitrary",)),
    )(x, targets)
    sums = jnp.sum(parts, axis=(0, 1))
    tss = jnp.maximum(sums[1], 1.0)
    return (sums[0] / tss * _GAIN_CLS + sums[2] / tss * _GAIN_BOX +
            sums[3] / tss * _GAIN_DFL)


# speedup vs baseline: 1.2269x; 1.2269x over previous
import jax
import jax.numpy as jnp
from jax.experimental import pallas as pl
from jax.experimental.pallas import tpu as pltpu

_NC = 80
_DFL = 16
_CH = 4 * _DFL + _NC
_STRIDE = 8.0
_EPS = 1e-9
_GAIN_CLS = 0.5
_GAIN_BOX = 7.5
_GAIN_DFL = 1.5
_B, _H, _W = 16, 80, 80
_A = _H * _W
_NT = 256
_CT = 64
_NCH = _NT // _CT
_BIG = 1e4
_HIGH = jax.lax.Precision.HIGHEST


def _atan(x):
    s = jnp.sign(x)
    a = jnp.abs(x)
    big = a > 2.414213562373095
    mid = a > 0.4142135623730951
    safe = jnp.maximum(a, 1e-30)
    r = jnp.where(big, -1.0 / safe, jnp.where(mid, (a - 1.0) / (a + 1.0), a))
    y0 = jnp.where(big, jnp.float32(jnp.pi / 2),
                   jnp.where(mid, jnp.float32(jnp.pi / 4), jnp.float32(0.0)))
    z = r * r
    p = (((8.05374449538e-2 * z - 1.38776856032e-1) * z + 1.99777106478e-1)
         * z - 3.33329491539e-1) * z * r + r
    return s * (y0 + p)


def _loss_kernel(x_ref, t_ref, out_ref, gt_ref, sov_ref, al_ref):
    b = pl.program_id(0)
    f32 = jnp.float32
    tgt = t_ref[...]

    xT = jnp.transpose(x_ref[0], (1, 0))

    al_iota = jax.lax.broadcasted_iota(jnp.int32, (1, _A), 1)
    axr = (al_iota % _W).astype(f32) + 0.5
    ayr = (al_iota // _W).astype(f32) + 0.5

    proj = jax.lax.broadcasted_iota(jnp.int32, (_DFL, _A), 0).astype(f32)
    pd = []
    for c in range(4):
        ch = xT[c * _DFL:(c + 1) * _DFL, :]
        m = jnp.max(ch, axis=0, keepdims=True)
        e = jnp.exp(ch - m)
        pd.append(jnp.sum(e * proj, axis=0, keepdims=True) /
                  jnp.sum(e, axis=0, keepdims=True))
    px1 = axr - pd[0]
    py1 = ayr - pd[1]
    px2 = axr + pd[2]
    py2 = ayr + pd[3]

    psT = xT[4 * _DFL:, :]
    scoresT = jax.nn.sigmoid(psT)

    timg = tgt[:, 0:1].astype(jnp.int32)
    mask = (timg == b).astype(f32)
    ti = jax.lax.broadcasted_iota(jnp.int32, (_NT, _NT), 0)
    tj = jax.lax.broadcasted_iota(jnp.int32, (_NT, _NT), 1)
    tril = (tj <= ti).astype(f32)
    cmask = jax.lax.dot_general(tril, mask, (((1,), (0,)), ((), ())),
                                precision=_HIGH)
    occ = cmask - 1.0
    perm = mask * (tj.astype(f32) == occ)
    gt = jax.lax.dot_general(perm, tgt[:, 1:6], (((0,), (0,)), ((), ())),
                             precision=_HIGH)
    size = _W * _STRIDE
    cx = gt[:, 1:2] * size
    cy = gt[:, 2:3] * size
    bw = gt[:, 3:4] * size
    bh = gt[:, 4:5] * size
    gx1 = cx - bw * 0.5
    gy1 = cy - bh * 0.5
    gx2 = cx + bw * 0.5
    gy2 = cy + bh * 0.5
    mgt = ((gx1 + gy1 + gx2 + gy2) > 0).astype(f32)
    gt_ref[:, 0:1] = gt[:, 0:1]
    gt_ref[:, 1:2] = gx1
    gt_ref[:, 2:3] = gy1
    gt_ref[:, 3:4] = gx2
    gt_ref[:, 4:5] = gy2
    gt_ref[:, 5:6] = mgt

    bx1 = px1 * _STRIDE
    by1 = py1 * _STRIDE
    bx2 = px2 * _STRIDE
    by2 = py2 * _STRIDE
    apix = (bx2 - bx1) * (by2 - by1)
    ax8 = axr * _STRIDE
    ay8 = ayr * _STRIDE

    ciota = jax.lax.broadcasted_iota(jnp.int32, (_CT, _NC), 1).astype(f32)
    riota = jax.lax.broadcasted_iota(jnp.int32, (_CT, 1), 0).astype(f32)
    neg = jnp.float32(-jnp.inf)

    def pass1(j, carry):
        fg, best_v, best_i = carry
        row = pl.ds(pl.multiple_of(j * _CT, _CT), _CT)
        g = gt_ref[row, :]
        lblj = g[:, 0:1]
        x1j = g[:, 1:2]
        y1j = g[:, 2:3]
        x2j = g[:, 3:4]
        y2j = g[:, 4:5]
        mgtj = g[:, 5:6]
        inter = (jnp.maximum(jnp.minimum(x2j, bx2) - jnp.maximum(x1j, bx1),
                             0.0) *
                 jnp.maximum(jnp.minimum(y2j, by2) - jnp.maximum(y1j, by1),
                             0.0))
        ag = (x2j - x1j) * (y2j - y1j)
        ovj = jnp.maximum(inter / (ag + apix - inter + _EPS), 0.0)
        selj = (ciota == lblj).astype(f32)
        bscj = jax.lax.dot_general(selj, scoresT, (((1,), (0,)), ((), ())),
                                   precision=_HIGH)
        o2 = ovj * ovj
        alj = jnp.sqrt(bscj) * (o2 * o2 * o2)
        m_inj = (jnp.minimum(jnp.minimum(ax8 - x1j, ay8 - y1j),
                             jnp.minimum(x2j - ax8, y2j - ay8))
                 > _EPS).astype(f32)
        metj = alj * m_inj * mgtj

        def extract(_, work):
            mx = jnp.max(work, axis=1, keepdims=True)
            return jnp.where(work >= mx, neg, work)

        work = jax.lax.fori_loop(0, 9, extract, metj)
        kth = jnp.max(work, axis=1, keepdims=True)
        mpj = jnp.logical_and(metj >= kth, metj > _EPS)
        fg = fg + jnp.sum(mpj.astype(f32), axis=0, keepdims=True)
        ridx = riota + (j * _CT).astype(f32)
        cmx = jnp.max(ovj, axis=0, keepdims=True)
        lidx = jnp.min(jnp.where(ovj == cmx, ridx, f32(_BIG)),
                       axis=0, keepdims=True)
        better = cmx > best_v
        best_v = jnp.where(better, cmx, best_v)
        best_i = jnp.where(better, lidx, best_i)
        sov_ref[row, :] = jnp.where(mpj, ovj, -ovj)
        al_ref[row, :] = alj
        return fg, best_v, best_i

    zero_r = jnp.zeros((1, _A), f32)
    fg, best_v, best_i = jax.lax.fori_loop(
        0, _NCH, pass1, (zero_r, zero_r - 1.0, zero_r))

    def pass2(j, carry):
        fg_cnt, tgt_min, t_lbl, tx1, ty1, tx2, ty2, norm = carry
        row = pl.ds(pl.multiple_of(j * _CT, _CT), _CT)
        g = gt_ref[row, :]
        sov = sov_ref[row, :]
        ovj = jnp.abs(sov)
        mpj = (sov > 0.0).astype(f32)
        alj = al_ref[row, :]
        ridx = riota + (j * _CT).astype(f32)
        ismj = (ridx == best_i).astype(f32)
        mposj = jnp.where(fg > 1.0, ismj, mpj)
        fg_cnt = fg_cnt + jnp.sum(mposj, axis=0, keepdims=True)
        tminj = jnp.min(jnp.where(mposj > 0.5, ridx, f32(_BIG)),
                        axis=0, keepdims=True)
        qj = (ridx == tminj).astype(f32)
        upd = tminj < tgt_min
        t_lbl = jnp.where(upd, jnp.sum(qj * g[:, 0:1], axis=0,
                                       keepdims=True), t_lbl)
        tx1 = jnp.where(upd, jnp.sum(qj * g[:, 1:2], axis=0,
                                     keepdims=True), tx1)
        ty1 = jnp.where(upd, jnp.sum(qj * g[:, 2:3], axis=0,
                                     keepdims=True), ty1)
        tx2 = jnp.where(upd, jnp.sum(qj * g[:, 3:4], axis=0,
                                     keepdims=True), tx2)
        ty2 = jnp.where(upd, jnp.sum(qj * g[:, 4:5], axis=0,
                                     keepdims=True), ty2)
        tgt_min = jnp.minimum(tgt_min, tminj)
        almj = alj * mposj
        pa = jnp.max(almj, axis=1, keepdims=True)
        po = jnp.max(ovj * mposj, axis=1, keepdims=True)
        norm = jnp.maximum(norm, jnp.max(almj * po / (pa + _EPS),
                                         axis=0, keepdims=True))
        return fg_cnt, tgt_min, t_lbl, tx1, ty1, tx2, ty2, norm

    fg_cnt, tgt_min, t_lbl, tx1, ty1, tx2, ty2, norm = jax.lax.fori_loop(
        0, _NCH, pass2,
        (zero_r, zero_r + f32(_BIG), zero_r, zero_r, zero_r, zero_r, zero_r,
         zero_r))
    no_hit = tgt_min >= f32(_BIG)
    t_lbl = jnp.where(no_hit, zero_r + gt_ref[0:1, 0:1], t_lbl)
    tx1 = jnp.where(no_hit, zero_r + gt_ref[0:1, 1:2], tx1)
    ty1 = jnp.where(no_hit, zero_r + gt_ref[0:1, 2:3], ty1)
    tx2 = jnp.where(no_hit, zero_r + gt_ref[0:1, 3:4], tx2)
    ty2 = jnp.where(no_hit, zero_r + gt_ref[0:1, 4:5], ty2)
    fgm = (fg_cnt > 0).astype(f32)
    weight = norm * fgm

    base = jnp.sum(jnp.maximum(psT, 0.0) + jnp.log1p(jnp.exp(-jnp.abs(psT))),
                   keepdims=True)
    crow = jax.lax.broadcasted_iota(jnp.int32, (_NC, _A), 0).astype(f32)
    ps_at = jnp.sum(jnp.where(crow == t_lbl, psT, 0.0),
                    axis=0, keepdims=True)
    bce_sum = base - jnp.sum(weight * ps_at, keepdims=True)

    tss = jnp.sum(weight, keepdims=True)

    e7 = 1e-7
    qx1 = tx1 / _STRIDE
    qy1 = ty1 / _STRIDE
    qx2 = tx2 / _STRIDE
    qy2 = ty2 / _STRIDE
    w1 = px2 - px1
    h1 = py2 - py1
    w2 = qx2 - qx1
    h2 = qy2 - qy1
    cinter = (jnp.maximum(jnp.minimum(px2, qx2) - jnp.maximum(px1, qx1), 0.0) *
              jnp.maximum(jnp.minimum(py2, qy2) - jnp.maximum(py1, qy1), 0.0))
    cunion = w1 * h1 + w2 * h2 - cinter + e7
    ciou_i = cinter / cunion
    cw = jnp.maximum(px2, qx2) - jnp.minimum(px1, qx1)
    chh = jnp.maximum(py2, qy2) - jnp.minimum(py1, qy1)
    c2 = cw * cw + chh * chh + e7
    rho2 = ((qx1 + qx2 - px1 - px2) ** 2 + (qy1 + qy2 - py1 - py2) ** 2) / 4.0
    v = (4.0 / (jnp.pi ** 2)) * (_atan(w2 / (h2 + e7)) -
                                 _atan(w1 / (h1 + e7))) ** 2
    alpha_t = v / (v - ciou_i + 1.0 + e7)
    iou_full = ciou_i - (rho2 / c2 + v * alpha_t)
    box_sum = jnp.sum((1.0 - iou_full) * weight, keepdims=True)

    tvals = (jnp.clip(axr - qx1, 0.0, _DFL - 1.01),
             jnp.clip(ayr - qy1, 0.0, _DFL - 1.01),
             jnp.clip(qx2 - axr, 0.0, _DFL - 1.01),
             jnp.clip(qy2 - ayr, 0.0, _DFL - 1.01))
    dfl_acc = jnp.zeros((1, _A), dtype=f32)
    for c in range(4):
        ch = xT[c * _DFL:(c + 1) * _DFL, :]
        m = jnp.max(ch, axis=0, keepdims=True)
        sh = ch - m
        logsm = sh - jnp.log(jnp.sum(jnp.exp(sh), axis=0, keepdims=True))
        tv = tvals[c]
        tl = jnp.floor(tv)
        wl = (tl + 1.0) - tv
        wr = 1.0 - wl
        trc = jnp.minimum(tl + 1.0, f32(_DFL - 1))
        ce_l = -jnp.sum(jnp.where(proj == tl, logsm, 0.0),
                        axis=0, keepdims=True)
        ce_r = -jnp.sum(jnp.where(proj == trc, logsm, 0.0),
                        axis=0, keepdims=True)
        dfl_acc = dfl_acc + ce_l * wl + ce_r * wr
    dfl_sum = jnp.sum(0.25 * dfl_acc * weight, keepdims=True)

    row = jnp.concatenate([bce_sum, tss, box_sum, dfl_sum], axis=1)
    out_ref[0] = row


def kernel(outputs, targets):
    x = outputs.reshape(_B, _A, _CH)
    parts = pl.pallas_call(
        _loss_kernel,
        grid=(_B,),
        in_specs=[
            pl.BlockSpec((1, _A, _CH), lambda b: (b, 0, 0)),
            pl.BlockSpec((_NT, 6), lambda b: (0, 0)),
        ],
        out_specs=pl.BlockSpec((1, 1, 4), lambda b: (b, 0, 0)),
        out_shape=jax.ShapeDtypeStruct((_B, 1, 4), jnp.float32),
        scratch_shapes=[
            pltpu.VMEM((_NT, 8), jnp.float32),
            pltpu.VMEM((_NT, _A), jnp.float32),
            pltpu.VMEM((_NT, _A), jnp.float32),
        ],
        compiler_params=pltpu.CompilerParams(
            dimension_semantics=("arbitrary",)),
    )(x, targets)
    sums = jnp.sum(parts, axis=(0, 1))
    tss = jnp.maximum(sums[1], 1.0)
    return (sums[0] / tss * _GAIN_CLS + sums[2] / tss * _GAIN_BOX +
            sums[3] / tss * _GAIN_DFL)


# speedup vs baseline: 1.2534x; 1.0216x over previous
import jax
import jax.numpy as jnp
from jax.experimental import pallas as pl
from jax.experimental.pallas import tpu as pltpu

_NC = 80
_DFL = 16
_CH = 4 * _DFL + _NC
_STRIDE = 8.0
_EPS = 1e-9
_GAIN_CLS = 0.5
_GAIN_BOX = 7.5
_GAIN_DFL = 1.5
_B, _H, _W = 16, 80, 80
_A = _H * _W
_NT = 256
_CT = 128
_NCH = _NT // _CT
_BIG = 1e4
_HIGH = jax.lax.Precision.HIGHEST


def _atan(x):
    s = jnp.sign(x)
    a = jnp.abs(x)
    big = a > 2.414213562373095
    mid = a > 0.4142135623730951
    safe = jnp.maximum(a, 1e-30)
    r = jnp.where(big, -1.0 / safe, jnp.where(mid, (a - 1.0) / (a + 1.0), a))
    y0 = jnp.where(big, jnp.float32(jnp.pi / 2),
                   jnp.where(mid, jnp.float32(jnp.pi / 4), jnp.float32(0.0)))
    z = r * r
    p = (((8.05374449538e-2 * z - 1.38776856032e-1) * z + 1.99777106478e-1)
         * z - 3.33329491539e-1) * z * r + r
    return s * (y0 + p)


def _loss_kernel(x_ref, t_ref, out_ref, gt_ref, sov_ref, al_ref):
    b = pl.program_id(0)
    f32 = jnp.float32
    tgt = t_ref[...]

    xT = jnp.transpose(x_ref[0], (1, 0))

    al_iota = jax.lax.broadcasted_iota(jnp.int32, (1, _A), 1)
    axr = (al_iota % _W).astype(f32) + 0.5
    ayr = (al_iota // _W).astype(f32) + 0.5

    proj = jax.lax.broadcasted_iota(jnp.int32, (_DFL, _A), 0).astype(f32)
    pd = []
    for c in range(4):
        ch = xT[c * _DFL:(c + 1) * _DFL, :]
        m = jnp.max(ch, axis=0, keepdims=True)
        e = jnp.exp(ch - m)
        pd.append(jnp.sum(e * proj, axis=0, keepdims=True) /
                  jnp.sum(e, axis=0, keepdims=True))
    px1 = axr - pd[0]
    py1 = ayr - pd[1]
    px2 = axr + pd[2]
    py2 = ayr + pd[3]

    psT = xT[4 * _DFL:, :]
    scoresT = jax.nn.sigmoid(psT)

    timg = tgt[:, 0:1].astype(jnp.int32)
    mask = (timg == b).astype(f32)
    ti = jax.lax.broadcasted_iota(jnp.int32, (_NT, _NT), 0)
    tj = jax.lax.broadcasted_iota(jnp.int32, (_NT, _NT), 1)
    tril = (tj <= ti).astype(f32)
    cmask = jax.lax.dot_general(tril, mask, (((1,), (0,)), ((), ())),
                                precision=_HIGH)
    occ = cmask - 1.0
    perm = mask * (tj.astype(f32) == occ)
    gt = jax.lax.dot_general(perm, tgt[:, 1:6], (((0,), (0,)), ((), ())),
                             precision=_HIGH)
    size = _W * _STRIDE
    cx = gt[:, 1:2] * size
    cy = gt[:, 2:3] * size
    bw = gt[:, 3:4] * size
    bh = gt[:, 4:5] * size
    gx1 = cx - bw * 0.5
    gy1 = cy - bh * 0.5
    gx2 = cx + bw * 0.5
    gy2 = cy + bh * 0.5
    mgt = ((gx1 + gy1 + gx2 + gy2) > 0).astype(f32)
    gt_ref[:, 0:1] = gt[:, 0:1]
    gt_ref[:, 1:2] = gx1
    gt_ref[:, 2:3] = gy1
    gt_ref[:, 3:4] = gx2
    gt_ref[:, 4:5] = gy2
    gt_ref[:, 5:6] = mgt

    bx1 = px1 * _STRIDE
    by1 = py1 * _STRIDE
    bx2 = px2 * _STRIDE
    by2 = py2 * _STRIDE
    apix = (bx2 - bx1) * (by2 - by1)
    ax8 = axr * _STRIDE
    ay8 = ayr * _STRIDE

    ciota = jax.lax.broadcasted_iota(jnp.int32, (_CT, _NC), 1).astype(f32)
    riota = jax.lax.broadcasted_iota(jnp.int32, (_CT, 1), 0).astype(f32)
    neg = jnp.float32(-jnp.inf)

    def pass1(j, carry):
        fg, best_v, best_i = carry
        row = pl.ds(pl.multiple_of(j * _CT, _CT), _CT)
        g = gt_ref[row, :]
        lblj = g[:, 0:1]
        x1j = g[:, 1:2]
        y1j = g[:, 2:3]
        x2j = g[:, 3:4]
        y2j = g[:, 4:5]
        mgtj = g[:, 5:6]
        inter = (jnp.maximum(jnp.minimum(x2j, bx2) - jnp.maximum(x1j, bx1),
                             0.0) *
                 jnp.maximum(jnp.minimum(y2j, by2) - jnp.maximum(y1j, by1),
                             0.0))
        ag = (x2j - x1j) * (y2j - y1j)
        ovj = jnp.maximum(inter / (ag + apix - inter + _EPS), 0.0)
        selj = (ciota == lblj).astype(f32)
        bscj = jax.lax.dot_general(selj, scoresT, (((1,), (0,)), ((), ())),
                                   precision=_HIGH)
        o2 = ovj * ovj
        alj = jnp.sqrt(bscj) * (o2 * o2 * o2)
        m_inj = (jnp.minimum(jnp.minimum(ax8 - x1j, ay8 - y1j),
                             jnp.minimum(x2j - ax8, y2j - ay8))
                 > _EPS).astype(f32)
        metj = alj * m_inj * mgtj

        def extract(_, work):
            mx = jnp.max(work, axis=1, keepdims=True)
            return jnp.where(work >= mx, neg, work)

        work = jax.lax.fori_loop(0, 9, extract, metj)
        kth = jnp.max(work, axis=1, keepdims=True)
        mpj = jnp.logical_and(metj >= kth, metj > _EPS)
        fg = fg + jnp.sum(mpj.astype(f32), axis=0, keepdims=True)
        ridx = riota + (j * _CT).astype(f32)
        cmx = jnp.max(ovj, axis=0, keepdims=True)
        lidx = jnp.min(jnp.where(ovj == cmx, ridx, f32(_BIG)),
                       axis=0, keepdims=True)
        better = cmx > best_v
        best_v = jnp.where(better, cmx, best_v)
        best_i = jnp.where(better, lidx, best_i)
        sov_ref[row, :] = jnp.where(mpj, ovj, -ovj)
        al_ref[row, :] = alj
        return fg, best_v, best_i

    zero_r = jnp.zeros((1, _A), f32)
    fg, best_v, best_i = jax.lax.fori_loop(
        0, _NCH, pass1, (zero_r, zero_r - 1.0, zero_r))

    def pass2(j, carry):
        fg_cnt, tgt_min, t_lbl, tx1, ty1, tx2, ty2, norm = carry
        row = pl.ds(pl.multiple_of(j * _CT, _CT), _CT)
        g = gt_ref[row, :]
        sov = sov_ref[row, :]
        ovj = jnp.abs(sov)
        mpj = (sov > 0.0).astype(f32)
        alj = al_ref[row, :]
        ridx = riota + (j * _CT).astype(f32)
        ismj = (ridx == best_i).astype(f32)
        mposj = jnp.where(fg > 1.0, ismj, mpj)
        fg_cnt = fg_cnt + jnp.sum(mposj, axis=0, keepdims=True)
        tminj = jnp.min(jnp.where(mposj > 0.5, ridx, f32(_BIG)),
                        axis=0, keepdims=True)
        qj = (ridx == tminj).astype(f32)
        upd = tminj < tgt_min
        t_lbl = jnp.where(upd, jnp.sum(qj * g[:, 0:1], axis=0,
                                       keepdims=True), t_lbl)
        tx1 = jnp.where(upd, jnp.sum(qj * g[:, 1:2], axis=0,
                                     keepdims=True), tx1)
        ty1 = jnp.where(upd, jnp.sum(qj * g[:, 2:3], axis=0,
                                     keepdims=True), ty1)
        tx2 = jnp.where(upd, jnp.sum(qj * g[:, 3:4], axis=0,
                                     keepdims=True), tx2)
        ty2 = jnp.where(upd, jnp.sum(qj * g[:, 4:5], axis=0,
                                     keepdims=True), ty2)
        tgt_min = jnp.minimum(tgt_min, tminj)
        almj = alj * mposj
        pa = jnp.max(almj, axis=1, keepdims=True)
        po = jnp.max(ovj * mposj, axis=1, keepdims=True)
        norm = jnp.maximum(norm, jnp.max(almj * po / (pa + _EPS),
                                         axis=0, keepdims=True))
        return fg_cnt, tgt_min, t_lbl, tx1, ty1, tx2, ty2, norm

    fg_cnt, tgt_min, t_lbl, tx1, ty1, tx2, ty2, norm = jax.lax.fori_loop(
        0, _NCH, pass2,
        (zero_r, zero_r + f32(_BIG), zero_r, zero_r, zero_r, zero_r, zero_r,
         zero_r))
    no_hit = tgt_min >= f32(_BIG)
    t_lbl = jnp.where(no_hit, zero_r + gt_ref[0:1, 0:1], t_lbl)
    tx1 = jnp.where(no_hit, zero_r + gt_ref[0:1, 1:2], tx1)
    ty1 = jnp.where(no_hit, zero_r + gt_ref[0:1, 2:3], ty1)
    tx2 = jnp.where(no_hit, zero_r + gt_ref[0:1, 3:4], tx2)
    ty2 = jnp.where(no_hit, zero_r + gt_ref[0:1, 4:5], ty2)
    fgm = (fg_cnt > 0).astype(f32)
    weight = norm * fgm

    base = jnp.sum(jnp.maximum(psT, 0.0) + jnp.log1p(jnp.exp(-jnp.abs(psT))),
                   keepdims=True)
    crow = jax.lax.broadcasted_iota(jnp.int32, (_NC, _A), 0).astype(f32)
    ps_at = jnp.sum(jnp.where(crow == t_lbl, psT, 0.0),
                    axis=0, keepdims=True)
    bce_sum = base - jnp.sum(weight * ps_at, keepdims=True)

    tss = jnp.sum(weight, keepdims=True)

    e7 = 1e-7
    qx1 = tx1 / _STRIDE
    qy1 = ty1 / _STRIDE
    qx2 = tx2 / _STRIDE
    qy2 = ty2 / _STRIDE
    w1 = px2 - px1
    h1 = py2 - py1
    w2 = qx2 - qx1
    h2 = qy2 - qy1
    cinter = (jnp.maximum(jnp.minimum(px2, qx2) - jnp.maximum(px1, qx1), 0.0) *
              jnp.maximum(jnp.minimum(py2, qy2) - jnp.maximum(py1, qy1), 0.0))
    cunion = w1 * h1 + w2 * h2 - cinter + e7
    ciou_i = cinter / cunion
    cw = jnp.maximum(px2, qx2) - jnp.minimum(px1, qx1)
    chh = jnp.maximum(py2, qy2) - jnp.minimum(py1, qy1)
    c2 = cw * cw + chh * chh + e7
    rho2 = ((qx1 + qx2 - px1 - px2) ** 2 + (qy1 + qy2 - py1 - py2) ** 2) / 4.0
    v = (4.0 / (jnp.pi ** 2)) * (_atan(w2 / (h2 + e7)) -
                                 _atan(w1 / (h1 + e7))) ** 2
    alpha_t = v / (v - ciou_i + 1.0 + e7)
    iou_full = ciou_i - (rho2 / c2 + v * alpha_t)
    box_sum = jnp.sum((1.0 - iou_full) * weight, keepdims=True)

    tvals = (jnp.clip(axr - qx1, 0.0, _DFL - 1.01),
             jnp.clip(ayr - qy1, 0.0, _DFL - 1.01),
             jnp.clip(qx2 - axr, 0.0, _DFL - 1.01),
             jnp.clip(qy2 - ayr, 0.0, _DFL - 1.01))
    dfl_acc = jnp.zeros((1, _A), dtype=f32)
    for c in range(4):
        ch = xT[c * _DFL:(c + 1) * _DFL, :]
        m = jnp.max(ch, axis=0, keepdims=True)
        sh = ch - m
        logsm = sh - jnp.log(jnp.sum(jnp.exp(sh), axis=0, keepdims=True))
        tv = tvals[c]
        tl = jnp.floor(tv)
        wl = (tl + 1.0) - tv
        wr = 1.0 - wl
        trc = jnp.minimum(tl + 1.0, f32(_DFL - 1))
        ce_l = -jnp.sum(jnp.where(proj == tl, logsm, 0.0),
                        axis=0, keepdims=True)
        ce_r = -jnp.sum(jnp.where(proj == trc, logsm, 0.0),
                        axis=0, keepdims=True)
        dfl_acc = dfl_acc + ce_l * wl + ce_r * wr
    dfl_sum = jnp.sum(0.25 * dfl_acc * weight, keepdims=True)

    row = jnp.concatenate([bce_sum, tss, box_sum, dfl_sum], axis=1)
    out_ref[0] = row


def kernel(outputs, targets):
    x = outputs.reshape(_B, _A, _CH)
    parts = pl.pallas_call(
        _loss_kernel,
        grid=(_B,),
        in_specs=[
            pl.BlockSpec((1, _A, _CH), lambda b: (b, 0, 0)),
            pl.BlockSpec((_NT, 6), lambda b: (0, 0)),
        ],
        out_specs=pl.BlockSpec((1, 1, 4), lambda b: (b, 0, 0)),
        out_shape=jax.ShapeDtypeStruct((_B, 1, 4), jnp.float32),
        scratch_shapes=[
            pltpu.VMEM((_NT, 8), jnp.float32),
            pltpu.VMEM((_NT, _A), jnp.float32),
            pltpu.VMEM((_NT, _A), jnp.float32),
        ],
        compiler_params=pltpu.CompilerParams(
            dimension_semantics=("arbitrary",)),
    )(x, targets)
    sums = jnp.sum(parts, axis=(0, 1))
    tss = jnp.maximum(sums[1], 1.0)
    return (sums[0] / tss * _GAIN_CLS + sums[2] / tss * _GAIN_BOX +
            sums[3] / tss * _GAIN_DFL)


# speedup vs baseline: 1.3045x; 1.0408x over previous
import jax
import jax.numpy as jnp
from jax.experimental import pallas as pl
from jax.experimental.pallas import tpu as pltpu

_NC = 80
_DFL = 16
_CH = 4 * _DFL + _NC
_STRIDE = 8.0
_EPS = 1e-9
_GAIN_CLS = 0.5
_GAIN_BOX = 7.5
_GAIN_DFL = 1.5
_B, _H, _W = 16, 80, 80
_A = _H * _W
_NT = 256
_CT = 256
_NCH = _NT // _CT
_BIG = 1e4
_HIGH = jax.lax.Precision.HIGHEST


def _atan(x):
    s = jnp.sign(x)
    a = jnp.abs(x)
    big = a > 2.414213562373095
    mid = a > 0.4142135623730951
    safe = jnp.maximum(a, 1e-30)
    r = jnp.where(big, -1.0 / safe, jnp.where(mid, (a - 1.0) / (a + 1.0), a))
    y0 = jnp.where(big, jnp.float32(jnp.pi / 2),
                   jnp.where(mid, jnp.float32(jnp.pi / 4), jnp.float32(0.0)))
    z = r * r
    p = (((8.05374449538e-2 * z - 1.38776856032e-1) * z + 1.99777106478e-1)
         * z - 3.33329491539e-1) * z * r + r
    return s * (y0 + p)


def _loss_kernel(x_ref, t_ref, out_ref, gt_ref, sov_ref, al_ref):
    b = pl.program_id(0)
    f32 = jnp.float32
    tgt = t_ref[...]

    xT = jnp.transpose(x_ref[0], (1, 0))

    al_iota = jax.lax.broadcasted_iota(jnp.int32, (1, _A), 1)
    axr = (al_iota % _W).astype(f32) + 0.5
    ayr = (al_iota // _W).astype(f32) + 0.5

    proj = jax.lax.broadcasted_iota(jnp.int32, (_DFL, _A), 0).astype(f32)
    pd = []
    for c in range(4):
        ch = xT[c * _DFL:(c + 1) * _DFL, :]
        m = jnp.max(ch, axis=0, keepdims=True)
        e = jnp.exp(ch - m)
        pd.append(jnp.sum(e * proj, axis=0, keepdims=True) /
                  jnp.sum(e, axis=0, keepdims=True))
    px1 = axr - pd[0]
    py1 = ayr - pd[1]
    px2 = axr + pd[2]
    py2 = ayr + pd[3]

    psT = xT[4 * _DFL:, :]
    scoresT = jax.nn.sigmoid(psT)

    timg = tgt[:, 0:1].astype(jnp.int32)
    mask = (timg == b).astype(f32)
    ti = jax.lax.broadcasted_iota(jnp.int32, (_NT, _NT), 0)
    tj = jax.lax.broadcasted_iota(jnp.int32, (_NT, _NT), 1)
    tril = (tj <= ti).astype(f32)
    cmask = jax.lax.dot_general(tril, mask, (((1,), (0,)), ((), ())),
                                precision=_HIGH)
    occ = cmask - 1.0
    perm = mask * (tj.astype(f32) == occ)
    gt = jax.lax.dot_general(perm, tgt[:, 1:6], (((0,), (0,)), ((), ())),
                             precision=_HIGH)
    size = _W * _STRIDE
    cx = gt[:, 1:2] * size
    cy = gt[:, 2:3] * size
    bw = gt[:, 3:4] * size
    bh = gt[:, 4:5] * size
    gx1 = cx - bw * 0.5
    gy1 = cy - bh * 0.5
    gx2 = cx + bw * 0.5
    gy2 = cy + bh * 0.5
    mgt = ((gx1 + gy1 + gx2 + gy2) > 0).astype(f32)
    gt_ref[:, 0:1] = gt[:, 0:1]
    gt_ref[:, 1:2] = gx1
    gt_ref[:, 2:3] = gy1
    gt_ref[:, 3:4] = gx2
    gt_ref[:, 4:5] = gy2
    gt_ref[:, 5:6] = mgt

    bx1 = px1 * _STRIDE
    by1 = py1 * _STRIDE
    bx2 = px2 * _STRIDE
    by2 = py2 * _STRIDE
    apix = (bx2 - bx1) * (by2 - by1)
    ax8 = axr * _STRIDE
    ay8 = ayr * _STRIDE

    ciota = jax.lax.broadcasted_iota(jnp.int32, (_CT, _NC), 1).astype(f32)
    riota = jax.lax.broadcasted_iota(jnp.int32, (_CT, 1), 0).astype(f32)
    neg = jnp.float32(-jnp.inf)

    def pass1(j, carry):
        fg, best_v, best_i = carry
        row = pl.ds(pl.multiple_of(j * _CT, _CT), _CT)
        g = gt_ref[row, :]
        lblj = g[:, 0:1]
        x1j = g[:, 1:2]
        y1j = g[:, 2:3]
        x2j = g[:, 3:4]
        y2j = g[:, 4:5]
        mgtj = g[:, 5:6]
        inter = (jnp.maximum(jnp.minimum(x2j, bx2) - jnp.maximum(x1j, bx1),
                             0.0) *
                 jnp.maximum(jnp.minimum(y2j, by2) - jnp.maximum(y1j, by1),
                             0.0))
        ag = (x2j - x1j) * (y2j - y1j)
        ovj = jnp.maximum(inter / (ag + apix - inter + _EPS), 0.0)
        selj = (ciota == lblj).astype(f32)
        bscj = jax.lax.dot_general(selj, scoresT, (((1,), (0,)), ((), ())),
                                   precision=_HIGH)
        o2 = ovj * ovj
        alj = jnp.sqrt(bscj) * (o2 * o2 * o2)
        m_inj = (jnp.minimum(jnp.minimum(ax8 - x1j, ay8 - y1j),
                             jnp.minimum(x2j - ax8, y2j - ay8))
                 > _EPS).astype(f32)
        metj = alj * m_inj * mgtj

        def extract(_, work):
            mx = jnp.max(work, axis=1, keepdims=True)
            return jnp.where(work >= mx, neg, work)

        work = jax.lax.fori_loop(0, 9, extract, metj)
        kth = jnp.max(work, axis=1, keepdims=True)
        mpj = jnp.logical_and(metj >= kth, metj > _EPS)
        fg = fg + jnp.sum(mpj.astype(f32), axis=0, keepdims=True)
        ridx = riota + (j * _CT).astype(f32)
        cmx = jnp.max(ovj, axis=0, keepdims=True)
        lidx = jnp.min(jnp.where(ovj == cmx, ridx, f32(_BIG)),
                       axis=0, keepdims=True)
        better = cmx > best_v
        best_v = jnp.where(better, cmx, best_v)
        best_i = jnp.where(better, lidx, best_i)
        sov_ref[row, :] = jnp.where(mpj, ovj, -ovj)
        al_ref[row, :] = alj
        return fg, best_v, best_i

    zero_r = jnp.zeros((1, _A), f32)
    fg, best_v, best_i = jax.lax.fori_loop(
        0, _NCH, pass1, (zero_r, zero_r - 1.0, zero_r))

    def pass2(j, carry):
        fg_cnt, tgt_min, t_lbl, tx1, ty1, tx2, ty2, norm = carry
        row = pl.ds(pl.multiple_of(j * _CT, _CT), _CT)
        g = gt_ref[row, :]
        sov = sov_ref[row, :]
        ovj = jnp.abs(sov)
        mpj = (sov > 0.0).astype(f32)
        alj = al_ref[row, :]
        ridx = riota + (j * _CT).astype(f32)
        ismj = (ridx == best_i).astype(f32)
        mposj = jnp.where(fg > 1.0, ismj, mpj)
        fg_cnt = fg_cnt + jnp.sum(mposj, axis=0, keepdims=True)
        tminj = jnp.min(jnp.where(mposj > 0.5, ridx, f32(_BIG)),
                        axis=0, keepdims=True)
        qj = (ridx == tminj).astype(f32)
        upd = tminj < tgt_min
        t_lbl = jnp.where(upd, jnp.sum(qj * g[:, 0:1], axis=0,
                                       keepdims=True), t_lbl)
        tx1 = jnp.where(upd, jnp.sum(qj * g[:, 1:2], axis=0,
                                     keepdims=True), tx1)
        ty1 = jnp.where(upd, jnp.sum(qj * g[:, 2:3], axis=0,
                                     keepdims=True), ty1)
        tx2 = jnp.where(upd, jnp.sum(qj * g[:, 3:4], axis=0,
                                     keepdims=True), tx2)
        ty2 = jnp.where(upd, jnp.sum(qj * g[:, 4:5], axis=0,
                                     keepdims=True), ty2)
        tgt_min = jnp.minimum(tgt_min, tminj)
        almj = alj * mposj
        pa = jnp.max(almj, axis=1, keepdims=True)
        po = jnp.max(ovj * mposj, axis=1, keepdims=True)
        norm = jnp.maximum(norm, jnp.max(almj * po / (pa + _EPS),
                                         axis=0, keepdims=True))
        return fg_cnt, tgt_min, t_lbl, tx1, ty1, tx2, ty2, norm

    fg_cnt, tgt_min, t_lbl, tx1, ty1, tx2, ty2, norm = jax.lax.fori_loop(
        0, _NCH, pass2,
        (zero_r, zero_r + f32(_BIG), zero_r, zero_r, zero_r, zero_r, zero_r,
         zero_r))
    no_hit = tgt_min >= f32(_BIG)
    t_lbl = jnp.where(no_hit, zero_r + gt_ref[0:1, 0:1], t_lbl)
    tx1 = jnp.where(no_hit, zero_r + gt_ref[0:1, 1:2], tx1)
    ty1 = jnp.where(no_hit, zero_r + gt_ref[0:1, 2:3], ty1)
    tx2 = jnp.where(no_hit, zero_r + gt_ref[0:1, 3:4], tx2)
    ty2 = jnp.where(no_hit, zero_r + gt_ref[0:1, 4:5], ty2)
    fgm = (fg_cnt > 0).astype(f32)
    weight = norm * fgm

    base = jnp.sum(jnp.maximum(psT, 0.0) + jnp.log1p(jnp.exp(-jnp.abs(psT))),
                   keepdims=True)
    crow = jax.lax.broadcasted_iota(jnp.int32, (_NC, _A), 0).astype(f32)
    ps_at = jnp.sum(jnp.where(crow == t_lbl, psT, 0.0),
                    axis=0, keepdims=True)
    bce_sum = base - jnp.sum(weight * ps_at, keepdims=True)

    tss = jnp.sum(weight, keepdims=True)

    e7 = 1e-7
    qx1 = tx1 / _STRIDE
    qy1 = ty1 / _STRIDE
    qx2 = tx2 / _STRIDE
    qy2 = ty2 / _STRIDE
    w1 = px2 - px1
    h1 = py2 - py1
    w2 = qx2 - qx1
    h2 = qy2 - qy1
    cinter = (jnp.maximum(jnp.minimum(px2, qx2) - jnp.maximum(px1, qx1), 0.0) *
              jnp.maximum(jnp.minimum(py2, qy2) - jnp.maximum(py1, qy1), 0.0))
    cunion = w1 * h1 + w2 * h2 - cinter + e7
    ciou_i = cinter / cunion
    cw = jnp.maximum(px2, qx2) - jnp.minimum(px1, qx1)
    chh = jnp.maximum(py2, qy2) - jnp.minimum(py1, qy1)
    c2 = cw * cw + chh * chh + e7
    rho2 = ((qx1 + qx2 - px1 - px2) ** 2 + (qy1 + qy2 - py1 - py2) ** 2) / 4.0
    v = (4.0 / (jnp.pi ** 2)) * (_atan(w2 / (h2 + e7)) -
                                 _atan(w1 / (h1 + e7))) ** 2
    alpha_t = v / (v - ciou_i + 1.0 + e7)
    iou_full = ciou_i - (rho2 / c2 + v * alpha_t)
    box_sum = jnp.sum((1.0 - iou_full) * weight, keepdims=True)

    tvals = (jnp.clip(axr - qx1, 0.0, _DFL - 1.01),
             jnp.clip(ayr - qy1, 0.0, _DFL - 1.01),
             jnp.clip(qx2 - axr, 0.0, _DFL - 1.01),
             jnp.clip(qy2 - ayr, 0.0, _DFL - 1.01))
    dfl_acc = jnp.zeros((1, _A), dtype=f32)
    for c in range(4):
        ch = xT[c * _DFL:(c + 1) * _DFL, :]
        m = jnp.max(ch, axis=0, keepdims=True)
        sh = ch - m
        logsm = sh - jnp.log(jnp.sum(jnp.exp(sh), axis=0, keepdims=True))
        tv = tvals[c]
        tl = jnp.floor(tv)
        wl = (tl + 1.0) - tv
        wr = 1.0 - wl
        trc = jnp.minimum(tl + 1.0, f32(_DFL - 1))
        ce_l = -jnp.sum(jnp.where(proj == tl, logsm, 0.0),
                        axis=0, keepdims=True)
        ce_r = -jnp.sum(jnp.where(proj == trc, logsm, 0.0),
                        axis=0, keepdims=True)
        dfl_acc = dfl_acc + ce_l * wl + ce_r * wr
    dfl_sum = jnp.sum(0.25 * dfl_acc * weight, keepdims=True)

    row = jnp.concatenate([bce_sum, tss, box_sum, dfl_sum], axis=1)
    out_ref[0] = row


def kernel(outputs, targets):
    x = outputs.reshape(_B, _A, _CH)
    parts = pl.pallas_call(
        _loss_kernel,
        grid=(_B,),
        in_specs=[
            pl.BlockSpec((1, _A, _CH), lambda b: (b, 0, 0)),
            pl.BlockSpec((_NT, 6), lambda b: (0, 0)),
        ],
        out_specs=pl.BlockSpec((1, 1, 4), lambda b: (b, 0, 0)),
        out_shape=jax.ShapeDtypeStruct((_B, 1, 4), jnp.float32),
        scratch_shapes=[
            pltpu.VMEM((_NT, 8), jnp.float32),
            pltpu.VMEM((_NT, _A), jnp.float32),
            pltpu.VMEM((_NT, _A), jnp.float32),
        ],
        compiler_params=pltpu.CompilerParams(
            dimension_semantics=("arbitrary",)),
    )(x, targets)
    sums = jnp.sum(parts, axis=(0, 1))
    tss = jnp.maximum(sums[1], 1.0)
    return (sums[0] / tss * _GAIN_CLS + sums[2] / tss * _GAIN_BOX +
            sums[3] / tss * _GAIN_DFL)


# speedup vs baseline: 1.3287x; 1.0186x over previous
import jax
import jax.numpy as jnp
from jax.experimental import pallas as pl
from jax.experimental.pallas import tpu as pltpu

_NC = 80
_DFL = 16
_CH = 4 * _DFL + _NC
_STRIDE = 8.0
_EPS = 1e-9
_GAIN_CLS = 0.5
_GAIN_BOX = 7.5
_GAIN_DFL = 1.5
_B, _H, _W = 16, 80, 80
_A = _H * _W
_NT = 256
_CT = 256
_NCH = _NT // _CT
_BIG = 1e4
_HIGH = jax.lax.Precision.HIGHEST


def _atan(x):
    s = jnp.sign(x)
    a = jnp.abs(x)
    big = a > 2.414213562373095
    mid = a > 0.4142135623730951
    safe = jnp.maximum(a, 1e-30)
    r = jnp.where(big, -1.0 / safe, jnp.where(mid, (a - 1.0) / (a + 1.0), a))
    y0 = jnp.where(big, jnp.float32(jnp.pi / 2),
                   jnp.where(mid, jnp.float32(jnp.pi / 4), jnp.float32(0.0)))
    z = r * r
    p = (((8.05374449538e-2 * z - 1.38776856032e-1) * z + 1.99777106478e-1)
         * z - 3.33329491539e-1) * z * r + r
    return s * (y0 + p)


def _loss_kernel(x_ref, t_ref, out_ref):
    b = pl.program_id(0)
    f32 = jnp.float32
    tgt = t_ref[...]

    xT = jnp.transpose(x_ref[0], (1, 0))

    al_iota = jax.lax.broadcasted_iota(jnp.int32, (1, _A), 1)
    axr = (al_iota % _W).astype(f32) + 0.5
    ayr = (al_iota // _W).astype(f32) + 0.5

    proj = jax.lax.broadcasted_iota(jnp.int32, (_DFL, _A), 0).astype(f32)
    pd = []
    for c in range(4):
        ch = xT[c * _DFL:(c + 1) * _DFL, :]
        m = jnp.max(ch, axis=0, keepdims=True)
        e = jnp.exp(ch - m)
        pd.append(jnp.sum(e * proj, axis=0, keepdims=True) /
                  jnp.sum(e, axis=0, keepdims=True))
    px1 = axr - pd[0]
    py1 = ayr - pd[1]
    px2 = axr + pd[2]
    py2 = ayr + pd[3]

    psT = xT[4 * _DFL:, :]
    scoresT = jax.nn.sigmoid(psT)

    timg = tgt[:, 0:1].astype(jnp.int32)
    mask = (timg == b).astype(f32)
    ti = jax.lax.broadcasted_iota(jnp.int32, (_NT, _NT), 0)
    tj = jax.lax.broadcasted_iota(jnp.int32, (_NT, _NT), 1)
    tril = (tj <= ti).astype(f32)
    cmask = jax.lax.dot_general(tril, mask, (((1,), (0,)), ((), ())),
                                precision=_HIGH)
    occ = cmask - 1.0
    perm = mask * (tj.astype(f32) == occ)
    gt = jax.lax.dot_general(perm, tgt[:, 1:6], (((0,), (0,)), ((), ())),
                             precision=_HIGH)
    size = _W * _STRIDE
    cx = gt[:, 1:2] * size
    cy = gt[:, 2:3] * size
    bw = gt[:, 3:4] * size
    bh = gt[:, 4:5] * size
    gx1 = cx - bw * 0.5
    gy1 = cy - bh * 0.5
    gx2 = cx + bw * 0.5
    gy2 = cy + bh * 0.5
    mgt = ((gx1 + gy1 + gx2 + gy2) > 0).astype(f32)

    bx1 = px1 * _STRIDE
    by1 = py1 * _STRIDE
    bx2 = px2 * _STRIDE
    by2 = py2 * _STRIDE
    apix = (bx2 - bx1) * (by2 - by1)
    ax8 = axr * _STRIDE
    ay8 = ayr * _STRIDE

    ciota = jax.lax.broadcasted_iota(jnp.int32, (_NT, _NC), 1).astype(f32)
    riota = jax.lax.broadcasted_iota(jnp.int32, (_NT, 1), 0).astype(f32)
    neg = jnp.float32(-jnp.inf)

    inter = (jnp.maximum(jnp.minimum(gx2, bx2) - jnp.maximum(gx1, bx1), 0.0) *
             jnp.maximum(jnp.minimum(gy2, by2) - jnp.maximum(gy1, by1), 0.0))
    ag = (gx2 - gx1) * (gy2 - gy1)
    ov = jnp.maximum(inter / (ag + apix - inter + _EPS), 0.0)
    sel = (ciota == gt[:, 0:1]).astype(f32)
    bsc = jax.lax.dot_general(sel, scoresT, (((1,), (0,)), ((), ())),
                              precision=_HIGH)
    o2 = ov * ov
    align = jnp.sqrt(bsc) * (o2 * o2 * o2)
    m_in = (jnp.minimum(jnp.minimum(ax8 - gx1, ay8 - gy1),
                        jnp.minimum(gx2 - ax8, gy2 - ay8)) > _EPS).astype(f32)
    metric = align * m_in * mgt

    def extract(_, work):
        mx = jnp.max(work, axis=1, keepdims=True)
        return jnp.where(work >= mx, neg, work)

    work = jax.lax.fori_loop(0, 9, extract, metric)
    kth = jnp.max(work, axis=1, keepdims=True)
    mp = jnp.logical_and(metric >= kth, metric > _EPS).astype(f32)
    fg = jnp.sum(mp, axis=0, keepdims=True)
    cmx = jnp.max(ov, axis=0, keepdims=True)
    best_i = jnp.min(jnp.where(ov == cmx, riota, f32(_BIG)),
                     axis=0, keepdims=True)

    is_max = (riota == best_i).astype(f32)
    mpos = jnp.where(fg > 1.0, is_max, mp)
    fg_cnt = jnp.sum(mpos, axis=0, keepdims=True)
    tgt_min = jnp.min(jnp.where(mpos > 0.5, riota, f32(_BIG)),
                      axis=0, keepdims=True)
    sel_t = jnp.where(tgt_min >= f32(_BIG), 0.0, tgt_min)
    q = (riota == sel_t).astype(f32)
    t_lbl = jnp.sum(q * gt[:, 0:1], axis=0, keepdims=True)
    tx1 = jnp.sum(q * gx1, axis=0, keepdims=True)
    ty1 = jnp.sum(q * gy1, axis=0, keepdims=True)
    tx2 = jnp.sum(q * gx2, axis=0, keepdims=True)
    ty2 = jnp.sum(q * gy2, axis=0, keepdims=True)
    alm = align * mpos
    pa = jnp.max(alm, axis=1, keepdims=True)
    po = jnp.max(ov * mpos, axis=1, keepdims=True)
    norm = jnp.max(alm * po / (pa + _EPS), axis=0, keepdims=True)
    zero_r = jnp.zeros((1, _A), f32)
    fgm = (fg_cnt > 0).astype(f32)
    weight = norm * fgm

    base = jnp.sum(jnp.maximum(psT, 0.0) + jnp.log1p(jnp.exp(-jnp.abs(psT))),
                   keepdims=True)
    crow = jax.lax.broadcasted_iota(jnp.int32, (_NC, _A), 0).astype(f32)
    ps_at = jnp.sum(jnp.where(crow == t_lbl, psT, 0.0),
                    axis=0, keepdims=True)
    bce_sum = base - jnp.sum(weight * ps_at, keepdims=True)

    tss = jnp.sum(weight, keepdims=True)

    e7 = 1e-7
    qx1 = tx1 / _STRIDE
    qy1 = ty1 / _STRIDE
    qx2 = tx2 / _STRIDE
    qy2 = ty2 / _STRIDE
    w1 = px2 - px1
    h1 = py2 - py1
    w2 = qx2 - qx1
    h2 = qy2 - qy1
    cinter = (jnp.maximum(jnp.minimum(px2, qx2) - jnp.maximum(px1, qx1), 0.0) *
              jnp.maximum(jnp.minimum(py2, qy2) - jnp.maximum(py1, qy1), 0.0))
    cunion = w1 * h1 + w2 * h2 - cinter + e7
    ciou_i = cinter / cunion
    cw = jnp.maximum(px2, qx2) - jnp.minimum(px1, qx1)
    chh = jnp.maximum(py2, qy2) - jnp.minimum(py1, qy1)
    c2 = cw * cw + chh * chh + e7
    rho2 = ((qx1 + qx2 - px1 - px2) ** 2 + (qy1 + qy2 - py1 - py2) ** 2) / 4.0
    v = (4.0 / (jnp.pi ** 2)) * (_atan(w2 / (h2 + e7)) -
                                 _atan(w1 / (h1 + e7))) ** 2
    alpha_t = v / (v - ciou_i + 1.0 + e7)
    iou_full = ciou_i - (rho2 / c2 + v * alpha_t)
    box_sum = jnp.sum((1.0 - iou_full) * weight, keepdims=True)

    tvals = (jnp.clip(axr - qx1, 0.0, _DFL - 1.01),
             jnp.clip(ayr - qy1, 0.0, _DFL - 1.01),
             jnp.clip(qx2 - axr, 0.0, _DFL - 1.01),
             jnp.clip(qy2 - ayr, 0.0, _DFL - 1.01))
    dfl_acc = jnp.zeros((1, _A), dtype=f32)
    for c in range(4):
        ch = xT[c * _DFL:(c + 1) * _DFL, :]
        m = jnp.max(ch, axis=0, keepdims=True)
        sh = ch - m
        logsm = sh - jnp.log(jnp.sum(jnp.exp(sh), axis=0, keepdims=True))
        tv = tvals[c]
        tl = jnp.floor(tv)
        wl = (tl + 1.0) - tv
        wr = 1.0 - wl
        trc = jnp.minimum(tl + 1.0, f32(_DFL - 1))
        ce_l = -jnp.sum(jnp.where(proj == tl, logsm, 0.0),
                        axis=0, keepdims=True)
        ce_r = -jnp.sum(jnp.where(proj == trc, logsm, 0.0),
                        axis=0, keepdims=True)
        dfl_acc = dfl_acc + ce_l * wl + ce_r * wr
    dfl_sum = jnp.sum(0.25 * dfl_acc * weight, keepdims=True)

    row = jnp.concatenate([bce_sum, tss, box_sum, dfl_sum], axis=1)
    out_ref[0] = row


def kernel(outputs, targets):
    x = outputs.reshape(_B, _A, _CH)
    parts = pl.pallas_call(
        _loss_kernel,
        grid=(_B,),
        in_specs=[
            pl.BlockSpec((1, _A, _CH), lambda b: (b, 0, 0)),
            pl.BlockSpec((_NT, 6), lambda b: (0, 0)),
        ],
        out_specs=pl.BlockSpec((1, 1, 4), lambda b: (b, 0, 0)),
        out_shape=jax.ShapeDtypeStruct((_B, 1, 4), jnp.float32),
        compiler_params=pltpu.CompilerParams(
            dimension_semantics=("arbitrary",)),
    )(x, targets)
    sums = jnp.sum(parts, axis=(0, 1))
    tss = jnp.maximum(sums[1], 1.0)
    return (sums[0] / tss * _GAIN_CLS + sums[2] / tss * _GAIN_BOX +
            sums[3] / tss * _GAIN_DFL)


# speedup vs baseline: 1.4698x; 1.1062x over previous
import jax
import jax.numpy as jnp
from jax.experimental import pallas as pl
from jax.experimental.pallas import tpu as pltpu

_NC = 80
_DFL = 16
_CH = 4 * _DFL + _NC
_STRIDE = 8.0
_EPS = 1e-9
_GAIN_CLS = 0.5
_GAIN_BOX = 7.5
_GAIN_DFL = 1.5
_B, _H, _W = 16, 80, 80
_A = _H * _W
_NT = 256
_CT = 256
_NCH = _NT // _CT
_BIG = 1e4
_HIGH = jax.lax.Precision.HIGHEST


def _atan(x):
    s = jnp.sign(x)
    a = jnp.abs(x)
    big = a > 2.414213562373095
    mid = a > 0.4142135623730951
    safe = jnp.maximum(a, 1e-30)
    r = jnp.where(big, -1.0 / safe, jnp.where(mid, (a - 1.0) / (a + 1.0), a))
    y0 = jnp.where(big, jnp.float32(jnp.pi / 2),
                   jnp.where(mid, jnp.float32(jnp.pi / 4), jnp.float32(0.0)))
    z = r * r
    p = (((8.05374449538e-2 * z - 1.38776856032e-1) * z + 1.99777106478e-1)
         * z - 3.33329491539e-1) * z * r + r
    return s * (y0 + p)


def _loss_kernel(x_ref, t_ref, out_ref):
    b = pl.program_id(0)
    f32 = jnp.float32
    tgt = t_ref[...]

    xT = jnp.transpose(x_ref[0], (1, 0))

    al_iota = jax.lax.broadcasted_iota(jnp.int32, (1, _A), 1)
    axr = (al_iota % _W).astype(f32) + 0.5
    ayr = (al_iota // _W).astype(f32) + 0.5

    proj = jax.lax.broadcasted_iota(jnp.int32, (_DFL, _A), 0).astype(f32)
    pd = []
    for c in range(4):
        ch = xT[c * _DFL:(c + 1) * _DFL, :]
        m = jnp.max(ch, axis=0, keepdims=True)
        e = jnp.exp(ch - m)
        pd.append(jnp.sum(e * proj, axis=0, keepdims=True) /
                  jnp.sum(e, axis=0, keepdims=True))
    px1 = axr - pd[0]
    py1 = ayr - pd[1]
    px2 = axr + pd[2]
    py2 = ayr + pd[3]

    psT = xT[4 * _DFL:, :]
    scoresT = jax.nn.sigmoid(psT)

    timg = tgt[:, 0:1].astype(jnp.int32)
    mask = (timg == b).astype(f32)
    ti = jax.lax.broadcasted_iota(jnp.int32, (_NT, _NT), 0)
    tj = jax.lax.broadcasted_iota(jnp.int32, (_NT, _NT), 1)
    tril = (tj <= ti).astype(f32)
    cmask = jax.lax.dot_general(tril, mask, (((1,), (0,)), ((), ())),
                                precision=_HIGH)
    occ = cmask - 1.0
    perm = mask * (tj.astype(f32) == occ)
    gt = jax.lax.dot_general(perm, tgt[:, 1:6], (((0,), (0,)), ((), ())),
                             precision=_HIGH)
    size = _W * _STRIDE
    cx = gt[:, 1:2] * size
    cy = gt[:, 2:3] * size
    bw = gt[:, 3:4] * size
    bh = gt[:, 4:5] * size
    gx1 = cx - bw * 0.5
    gy1 = cy - bh * 0.5
    gx2 = cx + bw * 0.5
    gy2 = cy + bh * 0.5
    mgt = ((gx1 + gy1 + gx2 + gy2) > 0).astype(f32)

    bx1 = px1 * _STRIDE
    by1 = py1 * _STRIDE
    bx2 = px2 * _STRIDE
    by2 = py2 * _STRIDE
    apix = (bx2 - bx1) * (by2 - by1)
    ax8 = axr * _STRIDE
    ay8 = ayr * _STRIDE

    ciota = jax.lax.broadcasted_iota(jnp.int32, (_NT, _NC), 1).astype(f32)
    riota = jax.lax.broadcasted_iota(jnp.int32, (_NT, 1), 0).astype(f32)
    neg = jnp.float32(-jnp.inf)

    inter = (jnp.maximum(jnp.minimum(gx2, bx2) - jnp.maximum(gx1, bx1), 0.0) *
             jnp.maximum(jnp.minimum(gy2, by2) - jnp.maximum(gy1, by1), 0.0))
    ag = (gx2 - gx1) * (gy2 - gy1)
    ov = jnp.maximum(inter / (ag + apix - inter + _EPS), 0.0)
    sel = (ciota == gt[:, 0:1]).astype(f32)
    bsc = jax.lax.dot_general(sel, scoresT, (((1,), (0,)), ((), ())),
                              precision=_HIGH)
    o2 = ov * ov
    align = jnp.sqrt(bsc) * (o2 * o2 * o2)
    m_in = (jnp.minimum(jnp.minimum(ax8 - gx1, ay8 - gy1),
                        jnp.minimum(gx2 - ax8, gy2 - ay8)) > _EPS).astype(f32)
    metric = align * m_in * mgt

    def extract(_, m):
        return jnp.max(jnp.where(metric >= m, neg, metric),
                       axis=1, keepdims=True)

    kth = jax.lax.fori_loop(
        0, 9, extract, jnp.max(metric, axis=1, keepdims=True))
    mp = jnp.logical_and(metric >= kth, metric > _EPS).astype(f32)
    fg = jnp.sum(mp, axis=0, keepdims=True)
    cmx = jnp.max(ov, axis=0, keepdims=True)
    best_i = jnp.min(jnp.where(ov == cmx, riota, f32(_BIG)),
                     axis=0, keepdims=True)

    is_max = (riota == best_i).astype(f32)
    mpos = jnp.where(fg > 1.0, is_max, mp)
    fg_cnt = jnp.sum(mpos, axis=0, keepdims=True)
    tgt_min = jnp.min(jnp.where(mpos > 0.5, riota, f32(_BIG)),
                      axis=0, keepdims=True)
    sel_t = jnp.where(tgt_min >= f32(_BIG), 0.0, tgt_min)
    q = (riota == sel_t).astype(f32)
    t_lbl = jnp.sum(q * gt[:, 0:1], axis=0, keepdims=True)
    tx1 = jnp.sum(q * gx1, axis=0, keepdims=True)
    ty1 = jnp.sum(q * gy1, axis=0, keepdims=True)
    tx2 = jnp.sum(q * gx2, axis=0, keepdims=True)
    ty2 = jnp.sum(q * gy2, axis=0, keepdims=True)
    alm = align * mpos
    pa = jnp.max(alm, axis=1, keepdims=True)
    po = jnp.max(ov * mpos, axis=1, keepdims=True)
    norm = jnp.max(alm * po / (pa + _EPS), axis=0, keepdims=True)
    zero_r = jnp.zeros((1, _A), f32)
    fgm = (fg_cnt > 0).astype(f32)
    weight = norm * fgm

    base = jnp.sum(jnp.maximum(psT, 0.0) + jnp.log1p(jnp.exp(-jnp.abs(psT))),
                   keepdims=True)
    crow = jax.lax.broadcasted_iota(jnp.int32, (_NC, _A), 0).astype(f32)
    ps_at = jnp.sum(jnp.where(crow == t_lbl, psT, 0.0),
                    axis=0, keepdims=True)
    bce_sum = base - jnp.sum(weight * ps_at, keepdims=True)

    tss = jnp.sum(weight, keepdims=True)

    e7 = 1e-7
    qx1 = tx1 / _STRIDE
    qy1 = ty1 / _STRIDE
    qx2 = tx2 / _STRIDE
    qy2 = ty2 / _STRIDE
    w1 = px2 - px1
    h1 = py2 - py1
    w2 = qx2 - qx1
    h2 = qy2 - qy1
    cinter = (jnp.maximum(jnp.minimum(px2, qx2) - jnp.maximum(px1, qx1), 0.0) *
              jnp.maximum(jnp.minimum(py2, qy2) - jnp.maximum(py1, qy1), 0.0))
    cunion = w1 * h1 + w2 * h2 - cinter + e7
    ciou_i = cinter / cunion
    cw = jnp.maximum(px2, qx2) - jnp.minimum(px1, qx1)
    chh = jnp.maximum(py2, qy2) - jnp.minimum(py1, qy1)
    c2 = cw * cw + chh * chh + e7
    rho2 = ((qx1 + qx2 - px1 - px2) ** 2 + (qy1 + qy2 - py1 - py2) ** 2) / 4.0
    v = (4.0 / (jnp.pi ** 2)) * (_atan(w2 / (h2 + e7)) -
                                 _atan(w1 / (h1 + e7))) ** 2
    alpha_t = v / (v - ciou_i + 1.0 + e7)
    iou_full = ciou_i - (rho2 / c2 + v * alpha_t)
    box_sum = jnp.sum((1.0 - iou_full) * weight, keepdims=True)

    tvals = (jnp.clip(axr - qx1, 0.0, _DFL - 1.01),
             jnp.clip(ayr - qy1, 0.0, _DFL - 1.01),
             jnp.clip(qx2 - axr, 0.0, _DFL - 1.01),
             jnp.clip(qy2 - ayr, 0.0, _DFL - 1.01))
    dfl_acc = jnp.zeros((1, _A), dtype=f32)
    for c in range(4):
        ch = xT[c * _DFL:(c + 1) * _DFL, :]
        m = jnp.max(ch, axis=0, keepdims=True)
        sh = ch - m
        logsm = sh - jnp.log(jnp.sum(jnp.exp(sh), axis=0, keepdims=True))
        tv = tvals[c]
        tl = jnp.floor(tv)
        wl = (tl + 1.0) - tv
        wr = 1.0 - wl
        trc = jnp.minimum(tl + 1.0, f32(_DFL - 1))
        ce_l = -jnp.sum(jnp.where(proj == tl, logsm, 0.0),
                        axis=0, keepdims=True)
        ce_r = -jnp.sum(jnp.where(proj == trc, logsm, 0.0),
                        axis=0, keepdims=True)
        dfl_acc = dfl_acc + ce_l * wl + ce_r * wr
    dfl_sum = jnp.sum(0.25 * dfl_acc * weight, keepdims=True)

    row = jnp.concatenate([bce_sum, tss, box_sum, dfl_sum], axis=1)
    out_ref[0] = row


def kernel(outputs, targets):
    x = outputs.reshape(_B, _A, _CH)
    parts = pl.pallas_call(
        _loss_kernel,
        grid=(_B,),
        in_specs=[
            pl.BlockSpec((1, _A, _CH), lambda b: (b, 0, 0)),
            pl.BlockSpec((_NT, 6), lambda b: (0, 0)),
        ],
        out_specs=pl.BlockSpec((1, 1, 4), lambda b: (b, 0, 0)),
        out_shape=jax.ShapeDtypeStruct((_B, 1, 4), jnp.float32),
        compiler_params=pltpu.CompilerParams(
            dimension_semantics=("arbitrary",)),
    )(x, targets)
    sums = jnp.sum(parts, axis=(0, 1))
    tss = jnp.maximum(sums[1], 1.0)
    return (sums[0] / tss * _GAIN_CLS + sums[2] / tss * _GAIN_BOX +
            sums[3] / tss * _GAIN_DFL)


# speedup vs baseline: 1.5032x; 1.0227x over previous
import jax
import jax.numpy as jnp
from jax.experimental import pallas as pl
from jax.experimental.pallas import tpu as pltpu

_NC = 80
_DFL = 16
_CH = 4 * _DFL + _NC
_STRIDE = 8.0
_EPS = 1e-9
_GAIN_CLS = 0.5
_GAIN_BOX = 7.5
_GAIN_DFL = 1.5
_B, _H, _W = 16, 80, 80
_A = _H * _W
_NT = 256
_CT = 256
_NCH = _NT // _CT
_BIG = 1e4
_HIGH = jax.lax.Precision.HIGHEST


def _atan(x):
    s = jnp.sign(x)
    a = jnp.abs(x)
    big = a > 2.414213562373095
    mid = a > 0.4142135623730951
    safe = jnp.maximum(a, 1e-30)
    r = jnp.where(big, -1.0 / safe, jnp.where(mid, (a - 1.0) / (a + 1.0), a))
    y0 = jnp.where(big, jnp.float32(jnp.pi / 2),
                   jnp.where(mid, jnp.float32(jnp.pi / 4), jnp.float32(0.0)))
    z = r * r
    p = (((8.05374449538e-2 * z - 1.38776856032e-1) * z + 1.99777106478e-1)
         * z - 3.33329491539e-1) * z * r + r
    return s * (y0 + p)


def _loss_kernel(x_ref, t_ref, out_ref):
    b = pl.program_id(0)
    f32 = jnp.float32
    tgt = t_ref[...]

    xT = jnp.transpose(x_ref[0], (1, 0))

    al_iota = jax.lax.broadcasted_iota(jnp.int32, (1, _A), 1)
    axr = (al_iota % _W).astype(f32) + 0.5
    ayr = (al_iota // _W).astype(f32) + 0.5

    proj = jax.lax.broadcasted_iota(jnp.int32, (_DFL, _A), 0).astype(f32)
    pd = []
    for c in range(4):
        ch = xT[c * _DFL:(c + 1) * _DFL, :]
        m = jnp.max(ch, axis=0, keepdims=True)
        e = jnp.exp(ch - m)
        pd.append(jnp.sum(e * proj, axis=0, keepdims=True) /
                  jnp.sum(e, axis=0, keepdims=True))
    px1 = axr - pd[0]
    py1 = ayr - pd[1]
    px2 = axr + pd[2]
    py2 = ayr + pd[3]

    psT = xT[4 * _DFL:, :]
    scoresT = jax.nn.sigmoid(psT)

    timg = tgt[:, 0:1].astype(jnp.int32)
    mask = (timg == b).astype(f32)
    ti = jax.lax.broadcasted_iota(jnp.int32, (_NT, _NT), 0)
    tj = jax.lax.broadcasted_iota(jnp.int32, (_NT, _NT), 1)
    tril = (tj <= ti).astype(f32)
    cmask = jax.lax.dot_general(tril, mask, (((1,), (0,)), ((), ())),
                                precision=_HIGH)
    occ = cmask - 1.0
    perm = mask * (tj.astype(f32) == occ)
    gt = jax.lax.dot_general(perm, tgt[:, 1:6], (((0,), (0,)), ((), ())),
                             precision=_HIGH)
    size = _W * _STRIDE
    cx = gt[:, 1:2] * size
    cy = gt[:, 2:3] * size
    bw = gt[:, 3:4] * size
    bh = gt[:, 4:5] * size
    gx1 = cx - bw * 0.5
    gy1 = cy - bh * 0.5
    gx2 = cx + bw * 0.5
    gy2 = cy + bh * 0.5
    mgt = ((gx1 + gy1 + gx2 + gy2) > 0).astype(f32)

    bx1 = px1 * _STRIDE
    by1 = py1 * _STRIDE
    bx2 = px2 * _STRIDE
    by2 = py2 * _STRIDE
    apix = (bx2 - bx1) * (by2 - by1)
    ax8 = axr * _STRIDE
    ay8 = ayr * _STRIDE

    ciota = jax.lax.broadcasted_iota(jnp.int32, (_NT, _NC), 1).astype(f32)
    riota = jax.lax.broadcasted_iota(jnp.int32, (_NT, 1), 0).astype(f32)
    neg = jnp.float32(-jnp.inf)

    inter = (jnp.maximum(jnp.minimum(gx2, bx2) - jnp.maximum(gx1, bx1), 0.0) *
             jnp.maximum(jnp.minimum(gy2, by2) - jnp.maximum(gy1, by1), 0.0))
    ag = (gx2 - gx1) * (gy2 - gy1)
    ov = jnp.maximum(inter / (ag + apix - inter + _EPS), 0.0)
    sel = (ciota == gt[:, 0:1]).astype(f32)
    bsc = jax.lax.dot_general(sel, scoresT, (((1,), (0,)), ((), ())),
                              precision=_HIGH)
    o2 = ov * ov
    align = jnp.sqrt(bsc) * (o2 * o2 * o2)
    m_in = (jnp.minimum(jnp.minimum(ax8 - gx1, ay8 - gy1),
                        jnp.minimum(gx2 - ax8, gy2 - ay8)) > _EPS).astype(f32)
    metric = align * m_in * mgt

    def extract(_, m):
        return jnp.max(jnp.where(metric >= m, neg, metric),
                       axis=1, keepdims=True)

    kth = jax.lax.fori_loop(
        0, 9, extract, jnp.max(metric, axis=1, keepdims=True))
    mp = jnp.logical_and(metric >= kth, metric > _EPS).astype(f32)
    fg = jnp.sum(mp, axis=0, keepdims=True)
    cmx = jnp.max(ov, axis=0, keepdims=True)
    best_i = jnp.min(jnp.where(ov == cmx, riota, f32(_BIG)),
                     axis=0, keepdims=True)

    is_max = (riota == best_i).astype(f32)
    mpos = jnp.where(fg > 1.0, is_max, mp)
    fg_cnt = jnp.sum(mpos, axis=0, keepdims=True)
    tgt_min = jnp.min(jnp.where(mpos > 0.5, riota, f32(_BIG)),
                      axis=0, keepdims=True)
    sel_t = jnp.where(tgt_min >= f32(_BIG), 0.0, tgt_min)
    q = (riota == sel_t).astype(f32)
    gvals = jnp.concatenate([gt[:, 0:1], gx1, gy1, gx2, gy2], axis=1)
    tv = jax.lax.dot_general(gvals, q, (((0,), (0,)), ((), ())),
                             precision=_HIGH)
    t_lbl = tv[0:1, :]
    tx1 = tv[1:2, :]
    ty1 = tv[2:3, :]
    tx2 = tv[3:4, :]
    ty2 = tv[4:5, :]
    alm = align * mpos
    pa = jnp.max(alm, axis=1, keepdims=True)
    po = jnp.max(ov * mpos, axis=1, keepdims=True)
    norm = jnp.max(alm * po / (pa + _EPS), axis=0, keepdims=True)
    zero_r = jnp.zeros((1, _A), f32)
    fgm = (fg_cnt > 0).astype(f32)
    weight = norm * fgm

    base = jnp.sum(jnp.maximum(psT, 0.0) + jnp.log1p(jnp.exp(-jnp.abs(psT))),
                   keepdims=True)
    crow = jax.lax.broadcasted_iota(jnp.int32, (_NC, _A), 0).astype(f32)
    ps_at = jnp.sum(jnp.where(crow == t_lbl, psT, 0.0),
                    axis=0, keepdims=True)
    bce_sum = base - jnp.sum(weight * ps_at, keepdims=True)

    tss = jnp.sum(weight, keepdims=True)

    e7 = 1e-7
    qx1 = tx1 / _STRIDE
    qy1 = ty1 / _STRIDE
    qx2 = tx2 / _STRIDE
    qy2 = ty2 / _STRIDE
    w1 = px2 - px1
    h1 = py2 - py1
    w2 = qx2 - qx1
    h2 = qy2 - qy1
    cinter = (jnp.maximum(jnp.minimum(px2, qx2) - jnp.maximum(px1, qx1), 0.0) *
              jnp.maximum(jnp.minimum(py2, qy2) - jnp.maximum(py1, qy1), 0.0))
    cunion = w1 * h1 + w2 * h2 - cinter + e7
    ciou_i = cinter / cunion
    cw = jnp.maximum(px2, qx2) - jnp.minimum(px1, qx1)
    chh = jnp.maximum(py2, qy2) - jnp.minimum(py1, qy1)
    c2 = cw * cw + chh * chh + e7
    rho2 = ((qx1 + qx2 - px1 - px2) ** 2 + (qy1 + qy2 - py1 - py2) ** 2) / 4.0
    v = (4.0 / (jnp.pi ** 2)) * (_atan(w2 / (h2 + e7)) -
                                 _atan(w1 / (h1 + e7))) ** 2
    alpha_t = v / (v - ciou_i + 1.0 + e7)
    iou_full = ciou_i - (rho2 / c2 + v * alpha_t)
    box_sum = jnp.sum((1.0 - iou_full) * weight, keepdims=True)

    tvals = (jnp.clip(axr - qx1, 0.0, _DFL - 1.01),
             jnp.clip(ayr - qy1, 0.0, _DFL - 1.01),
             jnp.clip(qx2 - axr, 0.0, _DFL - 1.01),
             jnp.clip(qy2 - ayr, 0.0, _DFL - 1.01))
    dfl_acc = jnp.zeros((1, _A), dtype=f32)
    for c in range(4):
        ch = xT[c * _DFL:(c + 1) * _DFL, :]
        m = jnp.max(ch, axis=0, keepdims=True)
        sh = ch - m
        logsm = sh - jnp.log(jnp.sum(jnp.exp(sh), axis=0, keepdims=True))
        tv = tvals[c]
        tl = jnp.floor(tv)
        wl = (tl + 1.0) - tv
        wr = 1.0 - wl
        trc = jnp.minimum(tl + 1.0, f32(_DFL - 1))
        ce_l = -jnp.sum(jnp.where(proj == tl, logsm, 0.0),
                        axis=0, keepdims=True)
        ce_r = -jnp.sum(jnp.where(proj == trc, logsm, 0.0),
                        axis=0, keepdims=True)
        dfl_acc = dfl_acc + ce_l * wl + ce_r * wr
    dfl_sum = jnp.sum(0.25 * dfl_acc * weight, keepdims=True)

    row = jnp.concatenate([bce_sum, tss, box_sum, dfl_sum], axis=1)
    out_ref[0] = row


def kernel(outputs, targets):
    x = outputs.reshape(_B, _A, _CH)
    parts = pl.pallas_call(
        _loss_kernel,
        grid=(_B,),
        in_specs=[
            pl.BlockSpec((1, _A, _CH), lambda b: (b, 0, 0)),
            pl.BlockSpec((_NT, 6), lambda b: (0, 0)),
        ],
        out_specs=pl.BlockSpec((1, 1, 4), lambda b: (b, 0, 0)),
        out_shape=jax.ShapeDtypeStruct((_B, 1, 4), jnp.float32),
        compiler_params=pltpu.CompilerParams(
            dimension_semantics=("arbitrary",)),
    )(x, targets)
    sums = jnp.sum(parts, axis=(0, 1))
    tss = jnp.maximum(sums[1], 1.0)
    return (sums[0] / tss * _GAIN_CLS + sums[2] / tss * _GAIN_BOX +
            sums[3] / tss * _GAIN_DFL)
